# baseline (speedup 1.0000x reference)
import functools

import jax
import jax.numpy as jnp
from jax.experimental import pallas as pl


def _gelu(x):
    return x * 0.5 * (1.0 + jax.lax.erf(x * 0.7071067811865476))


def _nodeproj_body(s_ref, w_ref, o_ref):
    o_ref[...] = jnp.dot(s_ref[...].astype(jnp.bfloat16),
                         w_ref[...].astype(jnp.bfloat16),
                         preferred_element_type=jnp.float32)


def _node_proj(s, w, tile=1000):
    n, d = s.shape
    hd = w.shape[1]
    grid = n // tile
    return pl.pallas_call(
        _nodeproj_body,
        grid=(grid,),
        in_specs=[
            pl.BlockSpec((tile, d), lambda i: (i, 0)),
            pl.BlockSpec((d, hd), lambda i: (0, 0)),
        ],
        out_specs=pl.BlockSpec((tile, hd), lambda i: (i, 0)),
        out_shape=jax.ShapeDtypeStruct((n, hd), jnp.float32),
    )(s, w)


def _edge_body(sg_ref, z_ref, w1z_ref, b1_ref, w2_ref, b2_ref, w3_ref, b3_ref,
               vw1_ref, vb1_ref, vw2_ref, vb2_ref, vw3_ref, vb3_ref,
               aw_ref, v_ref, gmax_ref):
    bf = jnp.bfloat16
    z = z_ref[...].astype(bf)
    h = _gelu(sg_ref[...] + jnp.dot(z, w1z_ref[...].astype(bf),
                                    preferred_element_type=jnp.float32)
              + b1_ref[...])
    h = _gelu(jnp.dot(h.astype(bf), w2_ref[...].astype(bf),
                      preferred_element_type=jnp.float32) + b2_ref[...])
    aw = (jnp.dot(h.astype(bf), w3_ref[...].astype(bf),
                  preferred_element_type=jnp.float32) + b3_ref[...])
    aw_ref[...] = aw

    v = _gelu(jnp.dot(z, vw1_ref[...].astype(bf),
                      preferred_element_type=jnp.float32) + vb1_ref[...])
    v = _gelu(jnp.dot(v.astype(bf), vw2_ref[...].astype(bf),
                      preferred_element_type=jnp.float32) + vb2_ref[...])
    v_ref[...] = (jnp.dot(v.astype(bf), vw3_ref[...].astype(bf),
                          preferred_element_type=jnp.float32) + vb3_ref[...])

    tile_max = jnp.max(aw, axis=0, keepdims=True)

    @pl.when(pl.program_id(0) == 0)
    def _init():
        gmax_ref[...] = jnp.full_like(gmax_ref, -jnp.inf)

    gmax_ref[...] = jnp.maximum(gmax_ref[...], tile_max)


def _edge_mlp(sg, z, w1z, b1, w2, b2, w3, b3, vw1, vb1, vw2, vb2, vw3, vb3,
              tile=640):
    e, d = sg.shape
    dp = z.shape[1]
    heads = w3.shape[1]
    grid = e // tile
    full = lambda i: (0, 0)
    return pl.pallas_call(
        _edge_body,
        grid=(grid,),
        in_specs=[
            pl.BlockSpec((tile, d), lambda i: (i, 0)),
            pl.BlockSpec((tile, dp), lambda i: (i, 0)),
            pl.BlockSpec(w1z.shape, full), pl.BlockSpec(b1.shape, full),
            pl.BlockSpec(w2.shape, full), pl.BlockSpec(b2.shape, full),
            pl.BlockSpec(w3.shape, full), pl.BlockSpec(b3.shape, full),
            pl.BlockSpec(vw1.shape, full), pl.BlockSpec(vb1.shape, full),
            pl.BlockSpec(vw2.shape, full), pl.BlockSpec(vb2.shape, full),
            pl.BlockSpec(vw3.shape, full), pl.BlockSpec(vb3.shape, full),
        ],
        out_specs=[
            pl.BlockSpec((tile, heads), lambda i: (i, 0)),
            pl.BlockSpec((tile, d), lambda i: (i, 0)),
            pl.BlockSpec((1, heads), full),
        ],
        out_shape=[
            jax.ShapeDtypeStruct((e, heads), jnp.float32),
            jax.ShapeDtypeStruct((e, d), jnp.float32),
            jax.ShapeDtypeStruct((1, heads), jnp.float32),
        ],
    )(sg, z, w1z, b1, w2, b2, w3, b3, vw1, vb1, vw2, vb2, vw3, vb3)


def _wval_body(attn_ref, v_ref, wo_ref, p_ref):
    v = v_ref[...].astype(jnp.bfloat16)
    attn = attn_ref[...]
    heads = attn.shape[1]
    acc = jnp.zeros(v.shape, jnp.float32)
    for h in range(heads):
        vp = jnp.dot(v, wo_ref[h].astype(jnp.bfloat16),
                     preferred_element_type=jnp.float32)
        acc = acc + attn[:, h:h + 1] * vp
    p_ref[...] = acc


def _weighted_proj(attn, v, wo, tile=640):
    e, d = v.shape
    heads = attn.shape[1]
    grid = e // tile
    return pl.pallas_call(
        _wval_body,
        grid=(grid,),
        in_specs=[
            pl.BlockSpec((tile, heads), lambda i: (i, 0)),
            pl.BlockSpec((tile, d), lambda i: (i, 0)),
            pl.BlockSpec(wo.shape, lambda i: (0, 0, 0)),
        ],
        out_specs=pl.BlockSpec((tile, d), lambda i: (i, 0)),
        out_shape=jax.ShapeDtypeStruct((e, d), jnp.float32),
    )(attn, v, wo)


def _final_body(oagg_ref, s_ref, ob_ref, k1_ref, c1_ref,
                fw1_ref, fb1_ref, fw2_ref, fb2_ref, k2_ref, c2_ref, out_ref):
    o = oagg_ref[...] + ob_ref[...]
    s1 = s_ref[...] + o * k1_ref[...] + c1_ref[...]
    f = _gelu(jnp.dot(s1.astype(jnp.bfloat16), fw1_ref[...].astype(jnp.bfloat16),
                      preferred_element_type=jnp.float32) + fb1_ref[...])
    f = (jnp.dot(f.astype(jnp.bfloat16), fw2_ref[...].astype(jnp.bfloat16),
                 preferred_element_type=jnp.float32) + fb2_ref[...])
    out_ref[...] = s1 + f * k2_ref[...] + c2_ref[...]


def _final_stage(oagg, s, ob, k1, c1, fw1, fb1, fw2, fb2, k2, c2, tile=1000):
    n, d = s.shape
    grid = n // tile
    full = lambda i: (0, 0)
    return pl.pallas_call(
        _final_body,
        grid=(grid,),
        in_specs=[
            pl.BlockSpec((tile, d), lambda i: (i, 0)),
            pl.BlockSpec((tile, d), lambda i: (i, 0)),
            pl.BlockSpec(ob.shape, full), pl.BlockSpec(k1.shape, full),
            pl.BlockSpec(c1.shape, full), pl.BlockSpec(fw1.shape, full),
            pl.BlockSpec(fb1.shape, full), pl.BlockSpec(fw2.shape, full),
            pl.BlockSpec(fb2.shape, full), pl.BlockSpec(k2.shape, full),
            pl.BlockSpec(c2.shape, full),
        ],
        out_specs=pl.BlockSpec((tile, d), lambda i: (i, 0)),
        out_shape=jax.ShapeDtypeStruct((n, d), jnp.float32),
    )(oagg, s, ob, k1, c1, fw1, fb1, fw2, fb2, k2, c2)


def kernel(s, z, aw_W1, aw_b1, aw_W2, aw_b2, aw_W3, aw_b3,
           av_W1, av_b1, av_W2, av_b2, av_W3, av_b3,
           ao_W, ao_b, bn1_g, bn1_b, bn1_m, bn1_v,
           ff_W1, ff_b1, ff_W2, ff_b2, bn2_g, bn2_b, bn2_m, bn2_v,
           edge_idx):
    n, d = s.shape
    e = z.shape[0]
    heads = aw_W3.shape[1]
    dst = edge_idx[1]

    row = lambda x: x.reshape(1, -1)

    su = _node_proj(s, aw_W1[:d])

    sg = jnp.take(su, dst, axis=0)

    aw, v, gmax = _edge_mlp(
        sg, z, aw_W1[d:], row(aw_b1), aw_W2, row(aw_b2), aw_W3, row(aw_b3),
        av_W1, row(av_b1), av_W2, row(av_b2), av_W3, row(av_b3))

    w = jnp.exp(aw - gmax)
    denom = jax.ops.segment_sum(w, dst, num_segments=n)
    attn = w * (1.0 / denom)[dst]

    wo = ao_W.reshape(heads, d, d)
    p = _weighted_proj(attn, v, wo)

    oagg = jax.ops.segment_sum(p, dst, num_segments=n)

    k1 = bn1_g / jnp.sqrt(bn1_v + 1e-5)
    c1 = bn1_b - bn1_m * k1
    k2 = bn2_g / jnp.sqrt(bn2_v + 1e-5)
    c2 = bn2_b - bn2_m * k2
    return _final_stage(oagg, s, row(ao_b), row(k1), row(c1),
                        ff_W1, row(ff_b1), ff_W2, row(ff_b2), row(k2), row(c2))


# speedup vs baseline: 1.6991x; 1.6991x over previous
import functools

import jax
import jax.numpy as jnp
from jax import lax
from jax.experimental import pallas as pl
from jax.experimental.pallas import tpu as pltpu
from jax.experimental.pallas import tpu_sc as plsc

_NC, _NS, _L = 2, 16, 16
_NW = _NC * _NS
_CK = 128


def _gelu(x):
    return x * 0.5 * (1.0 + jax.lax.erf(x * 0.7071067811865476))


def _sc_mesh():
    return plsc.VectorSubcoreMesh(core_axis_name="c", subcore_axis_name="s")


_SC_PARAMS = pltpu.CompilerParams(needs_layout_passes=False)


def _sc_gather(table, idx1d):
    (e,) = idx1d.shape
    R, W = table.shape
    C = e // _CK
    base_n, extra = C // _NW, C % _NW

    @functools.partial(
        pl.kernel,
        out_type=jax.ShapeDtypeStruct((e, W), jnp.float32),
        mesh=_sc_mesh(),
        scratch_types=[
            pltpu.VMEM((_CK,), jnp.int32),
            pltpu.VMEM((_CK, W), jnp.float32),
            pltpu.SemaphoreType.DMA,
        ],
    )
    def k(table_hbm, idx_hbm, out_hbm, idx_v, rows_v, sem):
        cid = lax.axis_index("c")
        sid = lax.axis_index("s")
        wid = sid * _NC + cid
        nch = base_n + jnp.where(wid < extra, 1, 0)

        def body(i, carry):
            c = wid + i * _NW
            pltpu.sync_copy(idx_hbm.at[pl.ds(c * _CK, _CK)], idx_v)
            pltpu.async_copy(table_hbm.at[idx_v], rows_v, sem).wait()
            pltpu.sync_copy(rows_v, out_hbm.at[pl.ds(c * _CK, _CK)])
            return carry

        lax.fori_loop(0, nch, body, 0)

    return k(table, idx1d)


def _sc_segsum4(awT, gmax, dst, n):
    heads, e = awT.shape
    C = e // _CK
    base_n, extra = C // _NW, C % _NW

    @functools.partial(
        pl.kernel,
        out_type=jax.ShapeDtypeStruct((_NW, n * 4), jnp.float32),
        mesh=_sc_mesh(),
        scratch_types=[
            pltpu.VMEM((n * 4,), jnp.float32),
            pltpu.VMEM((heads, _CK), jnp.float32),
            pltpu.VMEM((_CK,), jnp.int32),
            pltpu.VMEM((heads, 128), jnp.float32),
        ],
        compiler_params=_SC_PARAMS,
    )
    def k(aw_hbm, gmax_hbm, dst_hbm, out_hbm, acc, abuf, dbuf, gsm):
        cid = lax.axis_index("c")
        sid = lax.axis_index("s")
        wid = sid * _NC + cid
        pltpu.sync_copy(gmax_hbm, gsm)

        def zb(j, carry):
            acc[pl.ds(j * _L, _L)] = jnp.zeros((_L,), jnp.float32)
            return carry

        lax.fori_loop(0, (n * 4) // _L, zb, 0)
        nch = base_n + jnp.where(wid < extra, 1, 0)

        def body(i, carry):
            c = wid + i * _NW
            pltpu.sync_copy(aw_hbm.at[:, pl.ds(c * _CK, _CK)], abuf)
            pltpu.sync_copy(dst_hbm.at[pl.ds(c * _CK, _CK)], dbuf)

            def jb(j, carry2):
                dvec = dbuf[pl.ds(j * _L, _L)]
                d4 = dvec << 2
                for h in range(heads):
                    gv = gsm[h, pl.ds(0, _L)]
                    wv = jnp.exp(abuf[h, pl.ds(j * _L, _L)] - gv)
                    plsc.addupdate_scatter(acc, [d4 + h], wv)
                return carry2

            lax.fori_loop(0, _CK // _L, jb, 0)
            return carry

        lax.fori_loop(0, nch, body, 0)
        pltpu.sync_copy(acc, out_hbm.at[wid])

    return k(awT, gmax, dst)


def _sc_attn(awT, gmax, rd1d, dst):
    heads, e = awT.shape
    (n4,) = rd1d.shape
    C = e // _CK
    base_n, extra = C // _NW, C % _NW

    @functools.partial(
        pl.kernel,
        out_type=jax.ShapeDtypeStruct((heads, e), jnp.float32),
        mesh=_sc_mesh(),
        scratch_types=[
            pltpu.VMEM((n4,), jnp.float32),
            pltpu.VMEM((heads, _CK), jnp.float32),
            pltpu.VMEM((heads, _CK), jnp.float32),
            pltpu.VMEM((_CK,), jnp.int32),
            pltpu.VMEM((heads, 128), jnp.float32),
        ],
        compiler_params=_SC_PARAMS,
    )
    def k(aw_hbm, gmax_hbm, rd_hbm, dst_hbm, out_hbm, rdt, abuf, obuf, dbuf,
          gsm):
        cid = lax.axis_index("c")
        sid = lax.axis_index("s")
        wid = sid * _NC + cid
        pltpu.sync_copy(gmax_hbm, gsm)
        pltpu.sync_copy(rd_hbm, rdt)
        nch = base_n + jnp.where(wid < extra, 1, 0)

        def body(i, carry):
            c = wid + i * _NW
            pltpu.sync_copy(aw_hbm.at[:, pl.ds(c * _CK, _CK)], abuf)
            pltpu.sync_copy(dst_hbm.at[pl.ds(c * _CK, _CK)], dbuf)

            def jb(j, carry2):
                dvec = dbuf[pl.ds(j * _L, _L)]
                d4 = dvec << 2
                for h in range(heads):
                    gv = gsm[h, pl.ds(0, _L)]
                    wv = jnp.exp(abuf[h, pl.ds(j * _L, _L)] - gv)
                    rdv = plsc.load_gather(rdt, [d4 + h])
                    obuf[h, pl.ds(j * _L, _L)] = wv * rdv
                return carry2

            lax.fori_loop(0, _CK // _L, jb, 0)
            pltpu.sync_copy(obuf, out_hbm.at[:, pl.ds(c * _CK, _CK)])
            return carry

        lax.fori_loop(0, nch, body, 0)

    return k(awT, gmax, rd1d, dst)


def _sc_scatter_cols(pT, dst, n, rows_per=8, chunk=1280):
    D, e = pT.shape
    nchunks = e // chunk

    @functools.partial(
        pl.kernel,
        out_type=jax.ShapeDtypeStruct((D, n), jnp.float32),
        mesh=_sc_mesh(),
        scratch_types=[
            pltpu.VMEM((rows_per, n), jnp.float32),
            pltpu.VMEM((rows_per, chunk), jnp.float32),
            pltpu.VMEM((chunk,), jnp.int32),
        ],
        compiler_params=_SC_PARAMS,
    )
    def k(p_hbm, dst_hbm, out_hbm, acc, pbuf, dbuf):
        cid = lax.axis_index("c")
        sid = lax.axis_index("s")
        wid = sid * _NC + cid
        r0 = wid * rows_per

        def zb(j, carry):
            for r in range(rows_per):
                acc[r, pl.ds(j * _L, _L)] = jnp.zeros((_L,), jnp.float32)
            return carry

        lax.fori_loop(0, n // _L, zb, 0)

        def body(i, carry):
            pltpu.sync_copy(p_hbm.at[pl.ds(r0, rows_per),
                                     pl.ds(i * chunk, chunk)], pbuf)
            pltpu.sync_copy(dst_hbm.at[pl.ds(i * chunk, chunk)], dbuf)

            def jb(j, carry2):
                dvec = dbuf[pl.ds(j * _L, _L)]
                for r in range(rows_per):
                    rv = jnp.full((_L,), r, jnp.int32)
                    pv = pbuf[r, pl.ds(j * _L, _L)]
                    plsc.addupdate_scatter(acc, [rv, dvec], pv)
                return carry2

            lax.fori_loop(0, chunk // _L, jb, 0)
            return carry

        lax.fori_loop(0, nchunks, body, 0)
        pltpu.sync_copy(acc, out_hbm.at[pl.ds(r0, rows_per)])

    return k(pT, dst)


def _nodeproj_body(s_ref, w_ref, o_ref):
    o_ref[...] = jnp.dot(s_ref[...].astype(jnp.bfloat16),
                         w_ref[...].astype(jnp.bfloat16),
                         preferred_element_type=jnp.float32)


def _node_proj(s, w, tile=1000):
    n, d = s.shape
    hd = w.shape[1]
    return pl.pallas_call(
        _nodeproj_body,
        grid=(n // tile,),
        in_specs=[
            pl.BlockSpec((tile, d), lambda i: (i, 0)),
            pl.BlockSpec((d, hd), lambda i: (0, 0)),
        ],
        out_specs=pl.BlockSpec((tile, hd), lambda i: (i, 0)),
        out_shape=jax.ShapeDtypeStruct((n, hd), jnp.float32),
    )(s, w)


def _edge_body(sg_ref, z_ref, w1z_ref, b1_ref, w2_ref, b2_ref, w3_ref, b3_ref,
               vw1_ref, vb1_ref, vw2_ref, vb2_ref, vw3_ref, vb3_ref,
               awt_ref, v_ref, gmax_ref):
    bf = jnp.bfloat16
    z = z_ref[...].astype(bf)
    h = _gelu(sg_ref[...] + jnp.dot(z, w1z_ref[...].astype(bf),
                                    preferred_element_type=jnp.float32)
              + b1_ref[...])
    h = _gelu(jnp.dot(h.astype(bf), w2_ref[...].astype(bf),
                      preferred_element_type=jnp.float32) + b2_ref[...])
    awt = lax.dot_general(w3_ref[...].astype(bf), h.astype(bf),
                          (((0,), (1,)), ((), ())),
                          preferred_element_type=jnp.float32) + b3_ref[...]
    awt_ref[...] = awt

    v = _gelu(jnp.dot(z, vw1_ref[...].astype(bf),
                      preferred_element_type=jnp.float32) + vb1_ref[...])
    v = _gelu(jnp.dot(v.astype(bf), vw2_ref[...].astype(bf),
                      preferred_element_type=jnp.float32) + vb2_ref[...])
    v_ref[...] = (jnp.dot(v.astype(bf), vw3_ref[...].astype(bf),
                          preferred_element_type=jnp.float32) + vb3_ref[...])

    tile_max = jnp.broadcast_to(jnp.max(awt, axis=1, keepdims=True),
                                gmax_ref.shape)

    @pl.when(pl.program_id(0) == 0)
    def _init():
        gmax_ref[...] = jnp.full_like(gmax_ref, -jnp.inf)

    gmax_ref[...] = jnp.maximum(gmax_ref[...], tile_max)


def _edge_mlp(sg, z, w1z, b1, w2, b2, w3, b3t, vw1, vb1, vw2, vb2, vw3, vb3,
              tile=640):
    e, d = sg.shape
    dp = z.shape[1]
    heads = w3.shape[1]
    full = lambda i: (0, 0)
    return pl.pallas_call(
        _edge_body,
        grid=(e // tile,),
        in_specs=[
            pl.BlockSpec((tile, d), lambda i: (i, 0)),
            pl.BlockSpec((tile, dp), lambda i: (i, 0)),
            pl.BlockSpec(w1z.shape, full), pl.BlockSpec(b1.shape, full),
            pl.BlockSpec(w2.shape, full), pl.BlockSpec(b2.shape, full),
            pl.BlockSpec(w3.shape, full), pl.BlockSpec(b3t.shape, full),
            pl.BlockSpec(vw1.shape, full), pl.BlockSpec(vb1.shape, full),
            pl.BlockSpec(vw2.shape, full), pl.BlockSpec(vb2.shape, full),
            pl.BlockSpec(vw3.shape, full), pl.BlockSpec(vb3.shape, full),
        ],
        out_specs=[
            pl.BlockSpec((heads, tile), lambda i: (0, i)),
            pl.BlockSpec((tile, d), lambda i: (i, 0)),
            pl.BlockSpec((heads, 128), full),
        ],
        out_shape=[
            jax.ShapeDtypeStruct((heads, e), jnp.float32),
            jax.ShapeDtypeStruct((e, d), jnp.float32),
            jax.ShapeDtypeStruct((heads, 128), jnp.float32),
        ],
    )(sg, z, w1z, b1, w2, b2, w3, b3t, vw1, vb1, vw2, vb2, vw3, vb3)


def _combine_body(parts_ref, rd_ref):
    s = jnp.sum(parts_ref[...], axis=0, keepdims=True)
    rd_ref[...] = 1.0 / s


def _combine_recip(parts):
    nw, m = parts.shape
    return pl.pallas_call(
        _combine_body,
        grid=(1,),
        in_specs=[pl.BlockSpec((nw, m), lambda i: (0, 0))],
        out_specs=pl.BlockSpec((1, m), lambda i: (0, 0)),
        out_shape=jax.ShapeDtypeStruct((1, m), jnp.float32),
    )(parts)


def _wval_body(at_ref, v_ref, wo_ref, pt_ref):
    at = at_ref[...]
    v = v_ref[...].astype(jnp.bfloat16)
    heads, t = at.shape
    d = v.shape[1]
    acc = jnp.zeros((d, t), jnp.float32)
    for h in range(heads):
        vpt = lax.dot_general(wo_ref[h].astype(jnp.bfloat16), v,
                              (((0,), (1,)), ((), ())),
                              preferred_element_type=jnp.float32)
        acc = acc + at[h:h + 1, :] * vpt
    pt_ref[...] = acc


def _weighted_proj(attnT, v, wo, tile=640):
    e, d = v.shape
    heads = attnT.shape[0]
    return pl.pallas_call(
        _wval_body,
        grid=(e // tile,),
        in_specs=[
            pl.BlockSpec((heads, tile), lambda i: (0, i)),
            pl.BlockSpec((tile, d), lambda i: (i, 0)),
            pl.BlockSpec(wo.shape, lambda i: (0, 0, 0)),
        ],
        out_specs=pl.BlockSpec((d, tile), lambda i: (0, i)),
        out_shape=jax.ShapeDtypeStruct((d, e), jnp.float32),
    )(attnT, v, wo)


def _transpose_body(xt_ref, ident_ref, x_ref):
    x_ref[...] = lax.dot_general(xt_ref[...], ident_ref[...],
                                 (((0,), (0,)), ((), ())),
                                 preferred_element_type=jnp.float32)


def _transpose_slab(xT, ident):
    d, n = xT.shape
    return pl.pallas_call(
        _transpose_body,
        grid=(1,),
        in_specs=[pl.BlockSpec((d, n), lambda i: (0, 0)),
                  pl.BlockSpec(ident.shape, lambda i: (0, 0))],
        out_specs=pl.BlockSpec((n, d), lambda i: (0, 0)),
        out_shape=jax.ShapeDtypeStruct((n, d), jnp.float32),
    )(xT, ident)


def _final_body(oagg_ref, s_ref, ob_ref, k1_ref, c1_ref,
                fw1_ref, fb1_ref, fw2_ref, fb2_ref, k2_ref, c2_ref, out_ref):
    o = oagg_ref[...] + ob_ref[...]
    s1 = s_ref[...] + o * k1_ref[...] + c1_ref[...]
    f = _gelu(jnp.dot(s1.astype(jnp.bfloat16), fw1_ref[...].astype(jnp.bfloat16),
                      preferred_element_type=jnp.float32) + fb1_ref[...])
    f = (jnp.dot(f.astype(jnp.bfloat16), fw2_ref[...].astype(jnp.bfloat16),
                 preferred_element_type=jnp.float32) + fb2_ref[...])
    out_ref[...] = s1 + f * k2_ref[...] + c2_ref[...]


def _final_stage(oagg, s, ob, k1, c1, fw1, fb1, fw2, fb2, k2, c2,
                 tile=1000):
    n, d = s.shape
    full = lambda i: (0, 0)
    return pl.pallas_call(
        _final_body,
        grid=(n // tile,),
        in_specs=[
            pl.BlockSpec((tile, d), lambda i: (i, 0)),
            pl.BlockSpec((tile, d), lambda i: (i, 0)),
            pl.BlockSpec(ob.shape, full), pl.BlockSpec(k1.shape, full),
            pl.BlockSpec(c1.shape, full), pl.BlockSpec(fw1.shape, full),
            pl.BlockSpec(fb1.shape, full), pl.BlockSpec(fw2.shape, full),
            pl.BlockSpec(fb2.shape, full), pl.BlockSpec(k2.shape, full),
            pl.BlockSpec(c2.shape, full),
        ],
        out_specs=pl.BlockSpec((tile, d), lambda i: (i, 0)),
        out_shape=jax.ShapeDtypeStruct((n, d), jnp.float32),
    )(oagg, s, ob, k1, c1, fw1, fb1, fw2, fb2, k2, c2)


def kernel(s, z, aw_W1, aw_b1, aw_W2, aw_b2, aw_W3, aw_b3,
           av_W1, av_b1, av_W2, av_b2, av_W3, av_b3,
           ao_W, ao_b, bn1_g, bn1_b, bn1_m, bn1_v,
           ff_W1, ff_b1, ff_W2, ff_b2, bn2_g, bn2_b, bn2_m, bn2_v,
           edge_idx):
    n, d = s.shape
    e = z.shape[0]
    heads = aw_W3.shape[1]
    dst = edge_idx[1]

    row = lambda x: x.reshape(1, -1)
    col = lambda x: x.reshape(-1, 1)

    su = _node_proj(s, aw_W1[:d])
    sg = _sc_gather(su, dst)
    awT, v, gmax = _edge_mlp(
        sg, z, aw_W1[d:], row(aw_b1), aw_W2, row(aw_b2), aw_W3, col(aw_b3),
        av_W1, row(av_b1), av_W2, row(av_b2), av_W3, row(av_b3))
    parts = _sc_segsum4(awT, gmax, dst, n)
    rd = _combine_recip(parts)
    attnT = _sc_attn(awT, gmax, rd.reshape(-1), dst)
    wo = ao_W.reshape(heads, d, d)
    pT = _weighted_proj(attnT, v, wo)
    oaggT = _sc_scatter_cols(pT, dst, n)
    ident = jnp.eye(d, dtype=jnp.float32)
    oagg = _transpose_slab(oaggT, ident)
    k1 = bn1_g / jnp.sqrt(bn1_v + 1e-5)
    c1 = bn1_b - bn1_m * k1
    k2 = bn2_g / jnp.sqrt(bn2_v + 1e-5)
    c2 = bn2_b - bn2_m * k2
    return _final_stage(oagg, s, row(ao_b), row(k1), row(c1),
                        ff_W1, row(ff_b1), ff_W2, row(ff_b2), row(k2), row(c2))
